# Initial kernel scaffold; baseline (speedup 1.0000x reference)
import jax
import jax.numpy as jnp
from jax.experimental import pallas as pl


def kernel(z, codebook):
    raise NotImplementedError("write your pallas kernel here")


# speedup vs baseline: 1.1567x; 1.1567x over previous
import functools

import jax
import jax.numpy as jnp
from jax import lax
from jax.experimental import pallas as pl
from jax.experimental.pallas import tpu as pltpu
from jax.experimental.pallas import tpu_sc as plsc

_BETA = 0.25
_BN = 512
_BK = 4096


def _argmin_body(z_ref, c_ref, idx_ref, dsum_ref):
    K = c_ref.shape[0]
    BN = z_ref.shape[0]
    zb = z_ref[...]
    z2 = jnp.sum(zb * zb, axis=1, keepdims=True)
    zb_bf = zb.astype(jnp.bfloat16)
    run_m = jnp.full((BN,), jnp.inf, dtype=jnp.float32)
    run_a = jnp.zeros((BN,), dtype=jnp.int32)
    for t in range(K // _BK):
        ck = c_ref[pl.ds(t * _BK, _BK), :]
        c2 = jnp.sum(ck * ck, axis=1)
        zc = lax.dot_general(zb_bf, ck.astype(jnp.bfloat16),
                             (((1,), (1,)), ((), ())),
                             preferred_element_type=jnp.float32)
        dist = (z2 + c2[None, :]) - 2.0 * zc
        m = jnp.min(dist, axis=1)
        ii = lax.broadcasted_iota(jnp.int32, dist.shape, 1)
        a = jnp.min(jnp.where(dist == m[:, None], ii, K), axis=1) + t * _BK
        upd = m < run_m
        run_a = jnp.where(upd, a, run_a)
        run_m = jnp.where(upd, m.astype(jnp.bfloat16).astype(jnp.float32),
                          run_m)
    idx_ref[...] = run_a

    @pl.when(pl.program_id(0) == 0)
    def _():
        dsum_ref[...] = jnp.zeros((1, 1), jnp.float32)

    dsum_ref[...] += jnp.sum(run_m).reshape(1, 1)


def _assign(z_flat, codebook):
    N, D = z_flat.shape
    K = codebook.shape[0]
    return pl.pallas_call(
        _argmin_body,
        grid=(N // _BN,),
        in_specs=[
            pl.BlockSpec((_BN, D), lambda i: (i, 0)),
            pl.BlockSpec((K, D), lambda i: (0, 0)),
        ],
        out_specs=[
            pl.BlockSpec((_BN,), lambda i: (i,)),
            pl.BlockSpec((1, 1), lambda i: (0, 0)),
        ],
        out_shape=[
            jax.ShapeDtypeStruct((N,), jnp.int32),
            jax.ShapeDtypeStruct((1, 1), jnp.float32),
        ],
    )(z_flat, codebook)


def _gather_rows(codebook, idx):
    K, D = codebook.shape
    B = idx.shape[0]
    info = plsc.get_sparse_core_info()
    nw = info.num_cores * info.num_subcores
    bpw = B // nw
    mesh = plsc.VectorSubcoreMesh(core_axis_name="c", subcore_axis_name="s")

    @functools.partial(
        pl.kernel, mesh=mesh,
        compiler_params=pltpu.CompilerParams(use_tc_tiling_on_sc=False),
        out_type=jax.ShapeDtypeStruct((B, D), jnp.float32),
        scratch_types=[
            pltpu.VMEM((bpw,), jnp.int32),
            pltpu.VMEM((bpw, D), jnp.float32),
            pltpu.SemaphoreType.DMA,
        ],
    )
    def k(table_hbm, idx_hbm, out_hbm, idx_v, rows_v, sem):
        wid = lax.axis_index("s") * info.num_cores + lax.axis_index("c")
        base = wid * bpw
        pltpu.sync_copy(idx_hbm.at[pl.ds(base, bpw)], idx_v)
        pltpu.async_copy(table_hbm.at[idx_v], rows_v, sem).wait()
        pltpu.sync_copy(rows_v, out_hbm.at[pl.ds(base, bpw)])

    return k(codebook, idx)


def kernel(z, codebook):
    b, d, h, w = z.shape
    zp = jnp.transpose(z, (0, 2, 3, 1))
    z_flat = zp.reshape(-1, d)
    indices, dsum = _assign(z_flat, codebook)
    z_q_flat = _gather_rows(codebook, indices)
    z_q_st = zp + (z_q_flat.reshape(b, h, w, d) - zp)
    q = jnp.transpose(z_q_st, (0, 3, 1, 2))
    loss = _BETA * (dsum[0, 0] / (z_flat.shape[0] * d))
    return q, loss, indices.reshape(b, h, w)
